# baseline (speedup 1.0000x reference)
import jax
import jax.numpy as jnp
from jax.experimental import pallas as pl
from jax.experimental.pallas import tpu as pltpu

NUM_GRAPHS = 128
HIDDEN = 256
BLOCK = 1024
CH = 32
NCH = BLOCK // CH
NEG_INF = float("-inf")


def _pool_kernel(starts, firsts, lasts, x_ref, sv_ref, wt_ref, b_ref,
                 o_ref, mx_ref, sm_ref, pm_ref, ps_ref):
    i = pl.program_id(0)
    nb = pl.num_programs(0)
    blk0 = i * BLOCK

    @pl.when(i == 0)
    def _():
        mx_ref[...] = jnp.full_like(mx_ref, NEG_INF)
        sm_ref[...] = jnp.zeros_like(sm_ref)

    for k in range(NCH):
        xk = x_ref[k * CH:(k + 1) * CH, :]
        pm_ref[pl.ds(k, 1), :] = jnp.max(xk, axis=0, keepdims=True)
        ps_ref[pl.ds(k, 1), :] = jnp.sum(xk, axis=0, keepdims=True)

    first = firsts[i]
    last = lasts[i]
    pm = pm_ref[...]
    ps = ps_ref[...]
    cidx = jax.lax.broadcasted_iota(jnp.int32, (NCH, HIDDEN), 0)

    def edge(base, r0, r1):
        xe = x_ref[pl.ds(base, CH), :]
        rid = jax.lax.broadcasted_iota(jnp.int32, (CH, HIDDEN), 0)
        m = (rid >= r0) & (rid < r1)
        emax = jnp.max(jnp.where(m, xe, NEG_INF), axis=0, keepdims=True)
        esum = jnp.sum(jnp.where(m, xe, 0.0), axis=0, keepdims=True)
        return emax, esum

    def body(s, carry):
        lo = jnp.maximum(starts[s], blk0) - blk0
        hi = jnp.minimum(starts[s + 1], blk0 + BLOCK) - blk0
        lcl = jnp.clip(lo // CH, 0, NCH - 1)
        lcr = jnp.clip((hi - 1) // CH, 0, NCH - 1)

        imask = (cidx > lcl) & (cidx < lcr)
        mxi = jnp.max(jnp.where(imask, pm, NEG_INF), axis=0, keepdims=True)
        smi = jnp.sum(jnp.where(imask, ps, 0.0), axis=0, keepdims=True)

        base_a = lcl * CH
        amax, asum = edge(base_a, lo - base_a, jnp.minimum(hi, base_a + CH)
                          - base_a)
        base_b = lcr * CH
        bmax, bsum = edge(base_b, jnp.maximum(lo, base_a + CH) - base_b,
                          hi - base_b)

        smax = jnp.maximum(jnp.maximum(mxi, amax), bmax)
        ssum = smi + asum + bsum
        mx_ref[pl.ds(s, 1), :] = jnp.maximum(mx_ref[pl.ds(s, 1), :], smax)
        sm_ref[pl.ds(s, 1), :] = sm_ref[pl.ds(s, 1), :] + ssum
        return carry

    jax.lax.fori_loop(first, last + 1, body, 0)

    @pl.when(i == nb - 1)
    def _():
        sv = sv_ref[...]
        counts = sv[1:NUM_GRAPHS + 1, :] - sv[:NUM_GRAPHS, :]
        mean = sm_ref[...] / jnp.maximum(counts, 1.0)
        comb = jnp.concatenate([mx_ref[...], mean], axis=1)
        o_ref[...] = jax.lax.dot_general(
            comb, wt_ref[...], (((1,), (0,)), ((), ())),
            preferred_element_type=jnp.float32) + b_ref[...]


@jax.jit
def kernel(x, batch, W, b):
    n, h = x.shape
    batch = batch.astype(jnp.int32)
    nb = pl.cdiv(n, BLOCK)
    npad = nb * BLOCK
    x = jnp.pad(x, ((0, npad - n), (0, 0)))
    segp = jnp.pad(batch, (0, npad - n), constant_values=NUM_GRAPHS)
    firsts = segp[::BLOCK]
    lasts = jnp.minimum(segp[BLOCK - 1::BLOCK], NUM_GRAPHS - 1)
    starts = jnp.searchsorted(batch, jnp.arange(NUM_GRAPHS + 1,
                                                dtype=jnp.int32)
                              ).astype(jnp.int32)
    sv = jnp.pad(starts.astype(jnp.float32),
                 (0, 7)).reshape(NUM_GRAPHS + 8, 1)
    wt = W.T
    b2 = b.reshape(1, h)

    out = pl.pallas_call(
        _pool_kernel,
        grid_spec=pltpu.PrefetchScalarGridSpec(
            num_scalar_prefetch=3,
            grid=(nb,),
            in_specs=[
                pl.BlockSpec((BLOCK, h), lambda i, *_: (i, 0)),
                pl.BlockSpec((NUM_GRAPHS + 8, 1), lambda i, *_: (0, 0)),
                pl.BlockSpec((2 * h, h), lambda i, *_: (0, 0)),
                pl.BlockSpec((1, h), lambda i, *_: (0, 0)),
            ],
            out_specs=pl.BlockSpec((NUM_GRAPHS, h), lambda i, *_: (0, 0)),
            scratch_shapes=[
                pltpu.VMEM((NUM_GRAPHS, h), jnp.float32),
                pltpu.VMEM((NUM_GRAPHS, h), jnp.float32),
                pltpu.VMEM((NCH, h), jnp.float32),
                pltpu.VMEM((NCH, h), jnp.float32),
            ],
        ),
        out_shape=jax.ShapeDtypeStruct((NUM_GRAPHS, h), jnp.float32),
    )(starts, firsts, lasts, x, sv, wt, b2)
    return out
